# baseline (speedup 1.0000x reference)
import functools

import jax
import jax.numpy as jnp
from jax import lax
from jax.experimental import pallas as pl
from jax.experimental.pallas import tpu as pltpu
from jax.experimental.pallas import tpu_sc as plsc

N = 200000
D = 128
H = 256
S = 20
K = 8192
M = 65536
R = 4

NC = 2
NS = 16
NW = NC * NS
CHUNK = 128

KC = K // (NW * CHUNK)
MC = M // (NW * CHUNK)


def _mesh():
    return plsc.VectorSubcoreMesh(core_axis_name="c", subcore_axis_name="s")


def _wid():
    return lax.axis_index("s") * NC + lax.axis_index("c")


def _gather_body(chunks, table, idx_hbm, out, idx_v, buf0, buf1, sem0, sem1):
    wid = _wid()
    pltpu.sync_copy(idx_hbm.at[wid], idx_v)
    bufs = (buf0, buf1)
    sems = (sem0, sem1)
    base = wid * chunks * CHUNK
    copies = [None, None]
    for c in range(chunks):
        p = c % 2
        copies[p] = pltpu.async_copy(table.at[idx_v.at[c]], bufs[p], sems[p])
        if c > 0:
            q = (c - 1) % 2
            copies[q].wait()
            pltpu.sync_copy(bufs[q], out.at[pl.ds(base + (c - 1) * CHUNK, CHUNK)])
    q = (chunks - 1) % 2
    copies[q].wait()
    pltpu.sync_copy(bufs[q], out.at[pl.ds(base + (chunks - 1) * CHUNK, CHUNK)])


def _make_gather(chunks):
    return functools.partial(
        pl.kernel,
        out_type=jax.ShapeDtypeStruct((NW * chunks * CHUNK, D), jnp.float32),
        mesh=_mesh(),
        scratch_types=[
            pltpu.VMEM((chunks, CHUNK), jnp.int32),
            pltpu.VMEM((CHUNK, D), jnp.float32),
            pltpu.VMEM((CHUNK, D), jnp.float32),
            pltpu.SemaphoreType.DMA,
            pltpu.SemaphoreType.DMA,
        ],
        name=f"sc_gather_{chunks}",
    )(functools.partial(_gather_body, chunks))


_gather_step = _make_gather(KC)
_gather_mask = _make_gather(MC)


def _scatter_body(table, idx_hbm, vals, idx_v, buf0, buf1, sem0, sem1):
    wid = _wid()
    pltpu.sync_copy(idx_hbm.at[wid], idx_v)
    bufs = (buf0, buf1)
    sems = (sem0, sem1)
    base = wid * KC * CHUNK
    copies = []
    for c in range(KC):
        p = c % 2
        pltpu.sync_copy(vals.at[pl.ds(base + c * CHUNK, CHUNK)], bufs[p])
        copies.append(pltpu.async_copy(bufs[p], table.at[idx_v.at[c]], sems[p]))
    for cp in copies:
        cp.wait()


_scatter_step = functools.partial(
    pl.kernel,
    out_type=(),
    mesh=_mesh(),
    scratch_types=[
        pltpu.VMEM((KC, CHUNK), jnp.int32),
        pltpu.VMEM((CHUNK, D), jnp.float32),
        pltpu.VMEM((CHUNK, D), jnp.float32),
        pltpu.SemaphoreType.DMA,
        pltpu.SemaphoreType.DMA,
    ],
    name="sc_scatter",
)(_scatter_body)


BM = 8192


def _mlp_body(r_ref, p_ref, w1_ref, b1_ref, w2_ref, b2_ref, o_ref):
    r = r_ref[0]
    w1 = w1_ref[r]
    h = jnp.dot(p_ref[...], w1, preferred_element_type=jnp.float32)
    h = jnp.maximum(h + b1_ref[r], 0.0)
    o = jnp.dot(h, w2_ref[r], preferred_element_type=jnp.float32)
    o_ref[...] = o + b2_ref[r]


def _mlp(r, parents, W1, b1, W2, b2):
    return pl.pallas_call(
        _mlp_body,
        grid=(K // BM,),
        in_specs=[
            pl.BlockSpec(memory_space=pltpu.SMEM),
            pl.BlockSpec((BM, D), lambda i: (i, 0)),
            pl.BlockSpec((R, D, H), lambda i: (0, 0, 0)),
            pl.BlockSpec((R, 1, H), lambda i: (0, 0, 0)),
            pl.BlockSpec((R, H, D), lambda i: (0, 0, 0)),
            pl.BlockSpec((R, 1, D), lambda i: (0, 0, 0)),
        ],
        out_specs=pl.BlockSpec((BM, D), lambda i: (i, 0)),
        out_shape=jax.ShapeDtypeStruct((K, D), jnp.float32),
        name="tc_rule_mlp",
    )(r, parents, W1, b1, W2, b2)


BL = 8192


def _loss_body(ew_ref, eb_ref, pw_ref, rows_ref, pos_ref, neg_ref, tgt_ref,
               loss_ref, posok_ref, negok_ref):
    i = pl.program_id(0)

    @pl.when(i == 0)
    def _():
        loss_ref[0, 0] = 0.0
        posok_ref[0, 0] = 0.0
        negok_ref[0, 0] = 0.0

    val = lax.dot_general(
        ew_ref[...], rows_ref[...],
        dimension_numbers=(((1,), (1,)), ((), ())),
        preferred_element_type=jnp.float32,
    ) + eb_ref[0]
    pos = pos_ref[0]
    neg = neg_ref[0]
    tgt = tgt_ref[0]
    ge = (val >= 0.0).astype(jnp.float32)
    posok_ref[0, 0] += jnp.sum(pos * ge)
    negok_ref[0, 0] += jnp.sum(neg * (1.0 - ge))
    sp_abs = jnp.log1p(jnp.exp(-jnp.abs(val)))
    sp_neg = sp_abs + jnp.maximum(-val, 0.0)
    sp_pos = sp_abs + jnp.maximum(val, 0.0)
    contrib = pw_ref[0] * tgt * sp_neg + (1.0 - tgt) * sp_pos
    loss_ref[0, 0] += jnp.sum((pos + neg) * contrib)


def _loss(eval_w, eval_b, pos_weight, rows, pos, neg, target):
    out_shape = jax.ShapeDtypeStruct((1, 1), jnp.float32)
    smem_out = pl.BlockSpec(memory_space=pltpu.SMEM)
    return pl.pallas_call(
        _loss_body,
        grid=(M // BL,),
        in_specs=[
            pl.BlockSpec((1, D), lambda i: (0, 0)),
            pl.BlockSpec(memory_space=pltpu.SMEM),
            pl.BlockSpec(memory_space=pltpu.SMEM),
            pl.BlockSpec((BL, D), lambda i: (i, 0)),
            pl.BlockSpec((1, 1, BL), lambda i: (i, 0, 0)),
            pl.BlockSpec((1, 1, BL), lambda i: (i, 0, 0)),
            pl.BlockSpec((1, 1, BL), lambda i: (i, 0, 0)),
        ],
        out_specs=(smem_out, smem_out, smem_out),
        out_shape=(out_shape, out_shape, out_shape),
        name="tc_eval_loss",
    )(eval_w.reshape(1, D), eval_b, pos_weight, rows,
      pos.reshape(M // BL, 1, BL), neg.reshape(M // BL, 1, BL),
      target.reshape(M // BL, 1, BL))


def kernel(vectors_init, W1, b1, W2, b2, eval_w, eval_b, pos_weight,
           pos, neg, target, ind_steps, pars_ind_steps, rule_steps, mask_idx):
    vec_ref = jax.new_ref(vectors_init)
    pars_idx = pars_ind_steps.reshape(S, NW, KC, CHUNK)
    ind_idx = ind_steps.reshape(S, NW, KC, CHUNK)
    mask_idx_r = mask_idx.reshape(NW, MC, CHUNK)
    b1r = b1.reshape(R, 1, H)
    b2r = b2.reshape(R, 1, D)
    for s in range(S):
        parents = _gather_step(vec_ref, pars_idx[s])
        out = _mlp(rule_steps[s].reshape(1), parents, W1, b1r, W2, b2r)
        _scatter_step(vec_ref, ind_idx[s], out)
    rows = _gather_mask(vec_ref, mask_idx_r)
    loss, posok, negok = _loss(eval_w, eval_b, pos_weight, rows, pos, neg, target)
    return loss.reshape(()), posok.reshape(()), negok.reshape(())


# speedup vs baseline: 1.0721x; 1.0721x over previous
import functools

import jax
import jax.numpy as jnp
from jax import lax
from jax.experimental import pallas as pl
from jax.experimental.pallas import tpu as pltpu
from jax.experimental.pallas import tpu_sc as plsc

N = 200000
D = 128
H = 256
S = 20
K = 8192
M = 65536
R = 4

NC = 2
NS = 16
NW = NC * NS
CHUNK = 128

KC = K // (NW * CHUNK)
MC = M // (NW * CHUNK)


def _mesh():
    return plsc.VectorSubcoreMesh(core_axis_name="c", subcore_axis_name="s")


def _wid():
    return lax.axis_index("s") * NC + lax.axis_index("c")


def _gather_body(chunks, nb, table, idx_hbm, out, idx_v, *rest):
    bufs = rest[:nb]
    gsems = rest[nb:2 * nb]
    wsems = rest[2 * nb:3 * nb]
    wid = _wid()
    pltpu.sync_copy(idx_hbm.at[wid], idx_v)
    base = wid * chunks * CHUNK
    g = [None] * nb
    w = [None] * nb
    for c in range(chunks):
        p = c % nb
        if w[p] is not None:
            w[p].wait()
        g[p] = pltpu.async_copy(table.at[idx_v.at[c]], bufs[p], gsems[p])
        if c >= nb - 1:
            cc = c - (nb - 1)
            q = cc % nb
            g[q].wait()
            w[q] = pltpu.async_copy(
                bufs[q], out.at[pl.ds(base + cc * CHUNK, CHUNK)], wsems[q])
    for cc in range(max(0, chunks - (nb - 1)), chunks):
        q = cc % nb
        g[q].wait()
        w[q] = pltpu.async_copy(
            bufs[q], out.at[pl.ds(base + cc * CHUNK, CHUNK)], wsems[q])
    for x in w:
        if x is not None:
            x.wait()


def _make_gather(chunks, nb):
    return functools.partial(
        pl.kernel,
        out_type=jax.ShapeDtypeStruct((NW * chunks * CHUNK, D), jnp.float32),
        mesh=_mesh(),
        scratch_types=(
            [pltpu.VMEM((chunks, CHUNK), jnp.int32)]
            + [pltpu.VMEM((CHUNK, D), jnp.float32) for _ in range(nb)]
            + [pltpu.SemaphoreType.DMA for _ in range(2 * nb)]
        ),
        name=f"sc_gather_{chunks}",
    )(functools.partial(_gather_body, chunks, nb))


_gather_step = _make_gather(KC, 2)
_gather_mask = _make_gather(MC, 4)


def _scatter_body(table, idx_hbm, vals, idx_v, buf0, buf1, rs0, rs1, ss0, ss1):
    wid = _wid()
    pltpu.sync_copy(idx_hbm.at[wid], idx_v)
    bufs = (buf0, buf1)
    rsems = (rs0, rs1)
    ssems = (ss0, ss1)
    base = wid * KC * CHUNK
    reads = [
        pltpu.async_copy(vals.at[pl.ds(base + c * CHUNK, CHUNK)], bufs[c], rsems[c])
        for c in range(KC)
    ]
    writes = []
    for c in range(KC):
        reads[c].wait()
        writes.append(pltpu.async_copy(bufs[c], table.at[idx_v.at[c]], ssems[c]))
    for cp in writes:
        cp.wait()


_scatter_step = functools.partial(
    pl.kernel,
    out_type=(),
    mesh=_mesh(),
    scratch_types=[
        pltpu.VMEM((KC, CHUNK), jnp.int32),
        pltpu.VMEM((CHUNK, D), jnp.float32),
        pltpu.VMEM((CHUNK, D), jnp.float32),
        pltpu.SemaphoreType.DMA,
        pltpu.SemaphoreType.DMA,
        pltpu.SemaphoreType.DMA,
        pltpu.SemaphoreType.DMA,
    ],
    name="sc_scatter",
)(_scatter_body)


BM = 4096


def _mlp_body(r_ref, p_ref, w1_ref, b1_ref, w2_ref, b2_ref, o_ref):
    r = r_ref[0]
    w1 = w1_ref[r]
    h = jnp.dot(p_ref[...], w1, preferred_element_type=jnp.float32)
    h = jnp.maximum(h + b1_ref[r], 0.0)
    o = jnp.dot(h, w2_ref[r], preferred_element_type=jnp.float32)
    o_ref[...] = o + b2_ref[r]


def _mlp(r, parents, W1, b1, W2, b2):
    return pl.pallas_call(
        _mlp_body,
        grid=(K // BM,),
        in_specs=[
            pl.BlockSpec(memory_space=pltpu.SMEM),
            pl.BlockSpec((BM, D), lambda i: (i, 0)),
            pl.BlockSpec((R, D, H), lambda i: (0, 0, 0)),
            pl.BlockSpec((R, 1, H), lambda i: (0, 0, 0)),
            pl.BlockSpec((R, H, D), lambda i: (0, 0, 0)),
            pl.BlockSpec((R, 1, D), lambda i: (0, 0, 0)),
        ],
        out_specs=pl.BlockSpec((BM, D), lambda i: (i, 0)),
        out_shape=jax.ShapeDtypeStruct((K, D), jnp.float32),
        name="tc_rule_mlp",
    )(r, parents, W1, b1, W2, b2)


BL = 8192


def _loss_body(ew_ref, eb_ref, pw_ref, rows_ref, pos_ref, neg_ref, tgt_ref,
               loss_ref, posok_ref, negok_ref):
    i = pl.program_id(0)

    @pl.when(i == 0)
    def _():
        loss_ref[0, 0] = 0.0
        posok_ref[0, 0] = 0.0
        negok_ref[0, 0] = 0.0

    val = lax.dot_general(
        ew_ref[...], rows_ref[...],
        dimension_numbers=(((1,), (1,)), ((), ())),
        preferred_element_type=jnp.float32,
    ) + eb_ref[0]
    pos = pos_ref[0]
    neg = neg_ref[0]
    tgt = tgt_ref[0]
    ge = (val >= 0.0).astype(jnp.float32)
    posok_ref[0, 0] += jnp.sum(pos * ge)
    negok_ref[0, 0] += jnp.sum(neg * (1.0 - ge))
    sp_abs = jnp.log1p(jnp.exp(-jnp.abs(val)))
    sp_neg = sp_abs + jnp.maximum(-val, 0.0)
    sp_pos = sp_abs + jnp.maximum(val, 0.0)
    contrib = pw_ref[0] * tgt * sp_neg + (1.0 - tgt) * sp_pos
    loss_ref[0, 0] += jnp.sum((pos + neg) * contrib)


def _loss(eval_w, eval_b, pos_weight, rows, pos, neg, target):
    out_shape = jax.ShapeDtypeStruct((1, 1), jnp.float32)
    smem_out = pl.BlockSpec(memory_space=pltpu.SMEM)
    return pl.pallas_call(
        _loss_body,
        grid=(M // BL,),
        in_specs=[
            pl.BlockSpec((1, D), lambda i: (0, 0)),
            pl.BlockSpec(memory_space=pltpu.SMEM),
            pl.BlockSpec(memory_space=pltpu.SMEM),
            pl.BlockSpec((BL, D), lambda i: (i, 0)),
            pl.BlockSpec((1, 1, BL), lambda i: (i, 0, 0)),
            pl.BlockSpec((1, 1, BL), lambda i: (i, 0, 0)),
            pl.BlockSpec((1, 1, BL), lambda i: (i, 0, 0)),
        ],
        out_specs=(smem_out, smem_out, smem_out),
        out_shape=(out_shape, out_shape, out_shape),
        name="tc_eval_loss",
    )(eval_w.reshape(1, D), eval_b, pos_weight, rows,
      pos.reshape(M // BL, 1, BL), neg.reshape(M // BL, 1, BL),
      target.reshape(M // BL, 1, BL))


def kernel(vectors_init, W1, b1, W2, b2, eval_w, eval_b, pos_weight,
           pos, neg, target, ind_steps, pars_ind_steps, rule_steps, mask_idx):
    vec_ref = jax.new_ref(vectors_init)
    pars_idx = pars_ind_steps.reshape(S, NW, KC, CHUNK)
    ind_idx = ind_steps.reshape(S, NW, KC, CHUNK)
    mask_idx_r = mask_idx.reshape(NW, MC, CHUNK)
    b1r = b1.reshape(R, 1, H)
    b2r = b2.reshape(R, 1, D)
    for s in range(S):
        parents = _gather_step(vec_ref, pars_idx[s])
        out = _mlp(rule_steps[s].reshape(1), parents, W1, b1r, W2, b2r)
        _scatter_step(vec_ref, ind_idx[s], out)
    rows = _gather_mask(vec_ref, mask_idx_r)
    loss, posok, negok = _loss(eval_w, eval_b, pos_weight, rows, pos, neg, target)
    return loss.reshape(()), posok.reshape(()), negok.reshape(())
